# baseline (speedup 1.0000x reference)
import jax
import jax.numpy as jnp
from jax import lax
from jax.experimental import pallas as pl
from jax.experimental.pallas import tpu as pltpu
from jax.experimental.pallas import tpu_sc as plsc

N = 131072
S = 4096
A = 8192
D = 128
NS = 16
L = 16

ROWS_PER_SUB = N // NS
CH = 64
NCHUNK = ROWS_PER_SUB // CH
IDROWS = ROWS_PER_SUB // 128

BN = 2048

_HI = lax.Precision.DEFAULT


def _mm(x, w):
    return lax.dot_general(x, w, (((1,), (0,)), ((), ())),
                           precision=_HI, preferred_element_type=jnp.float32)


def _conv_body(x0_ref, w0_ref, b0_ref, x1_ref, w1_ref, b1_ref, r0_ref, r1_ref):
    r0_ref[...] = jnp.maximum(_mm(x0_ref[...], w0_ref[...]) + b0_ref[...], 0.0)
    r1_ref[...] = jnp.maximum(_mm(x1_ref[...], w1_ref[...]) + b1_ref[...], 0.0)


def _make_conv(nrows, row0):
    o = row0 // BN
    return pl.pallas_call(
        _conv_body,
        grid=(nrows // BN,),
        in_specs=[
            pl.BlockSpec((BN, D), lambda i, o=o: (i + o, 0)),
            pl.BlockSpec((D, D), lambda i: (0, 0)),
            pl.BlockSpec((1, D), lambda i: (0, 0)),
            pl.BlockSpec((BN, D), lambda i, o=o: (i + o, 0)),
            pl.BlockSpec((D, D), lambda i: (0, 0)),
            pl.BlockSpec((1, D), lambda i: (0, 0)),
        ],
        out_specs=[pl.BlockSpec((BN, D), lambda i: (i, 0)),
                   pl.BlockSpec((BN, D), lambda i: (i, 0))],
        out_shape=[jax.ShapeDtypeStruct((nrows, D), jnp.float32),
                   jax.ShapeDtypeStruct((nrows, D), jnp.float32)],
    )


NA = 51200
NB = N - NA

_conv_a = _make_conv(NA, 0)
_conv_b = _make_conv(NB, NA)


def _make_segsum(nrows, idrow0):
    rows_per_sub = nrows // NS
    nchunk = rows_per_sub // CH

    def body(*refs):
        (r0, r1, sid0, aid0, sid1, aid1,
         rs0_o, rs1_o, ra0_o, ra1_o,
         buf0, buf1, sidv, aidv, ibuf, ibuf2, midx, mbuf,
         site_acc, all_acc, idinfo,
         gsem0, gsem1, ssem0, ssem1) = refs
        cid = lax.axis_index("c")
        sid = lax.axis_index("s")
        ssl = pl.ds(sid * (S // NS), S // NS)
        asl = pl.ds(sid * (A // NS), A // NS)
        lanes = lax.iota(jnp.int32, L)
        zeros16 = jnp.zeros((L,), jnp.float32)

        def zrow(i, carry):
            for c in range(D // L):
                buf0[i, pl.ds(c * L, L)] = zeros16
            return carry

        lax.fori_loop(0, CH, zrow, 0)

        for t in range(S // NS // CH):
            pltpu.sync_copy(
                buf0, site_acc.at[pl.ds(sid * (S // NS) + t * CH, CH)])
        for t in range(A // NS // CH):
            pltpu.sync_copy(
                buf0, all_acc.at[pl.ds(sid * (A // NS) + t * CH, CH)])

        pltpu.sync_copy(buf0.at[pl.ds(0, 1)], site_acc.at[pl.ds(S + sid, 1)])
        pltpu.sync_copy(buf0.at[pl.ds(0, 1)],
                        site_acc.at[pl.ds(S + NS + sid, 1)])
        pltpu.sync_copy(buf0.at[pl.ds(0, 1)], all_acc.at[pl.ds(A + sid, 1)])
        pltpu.sync_copy(buf0.at[pl.ds(0, 1)],
                        all_acc.at[pl.ds(A + NS + sid, 1)])

        def run(r_hbm, s_hbm, a_hbm):
            base_row = sid * rows_per_sub
            idbase = idrow0 + sid * nchunk
            pltpu.sync_copy(s_hbm.at[pl.ds(idbase, nchunk)], sidv)
            pltpu.sync_copy(a_hbm.at[pl.ds(idbase, nchunk)], aidv)

            first_s = sidv[0, pl.ds(0, L)][0]
            last_s = sidv[nchunk - 1, pl.ds(CH - L, L)][L - 1]
            first_a = aidv[0, pl.ds(0, L)][0]
            last_a = aidv[nchunk - 1, pl.ds(CH - L, L)][L - 1]
            info = jnp.where(lanes == 0, first_s,
                             jnp.where(lanes == 1, last_s,
                                       jnp.where(lanes == 2, first_a,
                                                 jnp.where(lanes == 3,
                                                           last_a, 0))))
            ibuf[...] = info
            pltpu.sync_copy(ibuf, idinfo.at[sid])

            def redirect(i, carry):
                for c in range(CH // L):
                    slc = pl.ds(c * L, L)
                    v = sidv[i, slc]
                    sidv[i, slc] = jnp.where(
                        v == first_s, S + sid,
                        jnp.where(v == last_s, S + NS + sid, v))
                    w = aidv[i, slc]
                    aidv[i, slc] = jnp.where(
                        w == first_a, A + sid,
                        jnp.where(w == last_a, A + NS + sid, w))
                return carry

            lax.fori_loop(0, nchunk, redirect, 0)

            bufs = (buf0, buf1)
            gsems = (gsem0, gsem1)
            ssems = (ssem0, ssem1)

            pltpu.async_copy(r_hbm.at[pl.ds(base_row, CH)], buf0, gsem0)
            pltpu.async_copy(r_hbm.at[pl.ds(base_row + CH, CH)], buf1, gsem1)

            def loop(i, carry):
                for b in range(2):
                    j = 2 * i + b
                    buf, gsem, ssem = bufs[b], gsems[b], ssems[b]
                    pltpu.make_async_copy(
                        r_hbm.at[pl.ds(base_row, CH)], buf, gsem).wait()
                    sidx = sidv.at[j]
                    aidx = aidv.at[j]
                    sa = pltpu.async_copy(buf, site_acc.at[sidx], ssem,
                                          add=True)
                    sb = pltpu.async_copy(buf, all_acc.at[aidx], ssem,
                                          add=True)
                    sa.wait()
                    sb.wait()

                    @pl.when(j + 2 < nchunk)
                    def _():
                        pltpu.async_copy(
                            r_hbm.at[pl.ds(base_row + (j + 2) * CH, CH)],
                            buf, gsem)
                return carry

            lax.fori_loop(0, nchunk // 2, loop, 0)

        @pl.when(cid == 0)
        def _():
            run(r0, sid0, aid0)

        @pl.when(cid == 1)
        def _():
            run(r1, sid1, aid1)

        plsc.subcore_barrier()

        @pl.when(sid == 0)
        def _():
            pltpu.sync_copy(idinfo, ibuf2)
            for acc, base, c0 in ((site_acc, S, 0), (all_acc, A, 2)):
                for blk, c in ((0, c0), (NS, c0 + 1)):
                    tgt = plsc.load_gather(
                        ibuf2, [lanes, jnp.full((L,), c, jnp.int32)])
                    midx[...] = tgt
                    pltpu.sync_copy(acc.at[pl.ds(base + blk, NS)], mbuf)
                    pltpu.sync_copy(mbuf, acc.at[midx], add=True)

        plsc.subcore_barrier()

        def flush(acc, out, sl):
            pltpu.sync_copy(acc.at[sl], out.at[sl])

        @pl.when(cid == 0)
        def _():
            flush(site_acc, rs0_o, ssl)
            flush(all_acc, ra0_o, asl)

        @pl.when(cid == 1)
        def _():
            flush(site_acc, rs1_o, ssl)
            flush(all_acc, ra1_o, asl)

    return pl.kernel(
        body,
        out_type=[jax.ShapeDtypeStruct((S, D), jnp.float32),
                  jax.ShapeDtypeStruct((S, D), jnp.float32),
                  jax.ShapeDtypeStruct((A, D), jnp.float32),
                  jax.ShapeDtypeStruct((A, D), jnp.float32)],
        mesh=plsc.VectorSubcoreMesh(core_axis_name="c", subcore_axis_name="s"),
        scratch_types=[
            pltpu.VMEM((CH, D), jnp.float32),
            pltpu.VMEM((CH, D), jnp.float32),
            pltpu.VMEM((nchunk, CH), jnp.int32),
            pltpu.VMEM((nchunk, CH), jnp.int32),
            pltpu.VMEM((L,), jnp.int32),
            pltpu.VMEM((NS, L), jnp.int32),
            pltpu.VMEM((L,), jnp.int32),
            pltpu.VMEM((NS, D), jnp.float32),
            pltpu.VMEM_SHARED((S + 2 * NS, D), jnp.float32),
            pltpu.VMEM_SHARED((A + 2 * NS, D), jnp.float32),
            pltpu.VMEM_SHARED((NS, L), jnp.int32),
            pltpu.SemaphoreType.DMA,
            pltpu.SemaphoreType.DMA,
            pltpu.SemaphoreType.DMA,
            pltpu.SemaphoreType.DMA,
        ],
        compiler_params=pltpu.CompilerParams(needs_layout_passes=False,
                                             use_tc_tiling_on_sc=False),
    )


_segsum_a = _make_segsum(NA, 0)
_segsum_b = _make_segsum(NB, NA // CH)


def _heads_body(rs0a, rs1a, ra0a, ra1a, rs0b, rs1b, ra0b, ra1b,
                wm0, wm1, bm, we0, be0, we1, be1,
                w2a, w2b, be2, meta_o, e0_o, e1_o, eh_o):
    rs0 = rs0a[...] + rs0b[...]
    rs1 = rs1a[...] + rs1b[...]
    ra0 = ra0a[...] + ra0b[...]
    ra1 = ra1a[...] + ra1b[...]
    m = _mm(rs0, wm0[...]) + _mm(rs1, wm1[...]) + bm[...]
    m = m - jnp.max(m, axis=1, keepdims=True)
    p = jnp.exp(m)
    meta_o[...] = p / jnp.sum(p, axis=1, keepdims=True)

    s0 = _mm(ra0, we0[...]) + be0[...]
    e0_o[...] = jnp.exp(s0 - jnp.max(s0))
    s1 = _mm(ra1, we1[...]) + be1[...]
    e1_o[...] = jnp.exp(s1 - jnp.max(s1))
    sh = _mm(ra0, w2a[...]) + _mm(ra1, w2b[...]) + be2[...]
    eh_o[...] = jnp.exp(sh - jnp.max(sh))


_heads = pl.pallas_call(
    _heads_body,
    out_shape=[jax.ShapeDtypeStruct((S, 3), jnp.float32),
               jax.ShapeDtypeStruct((A, 1), jnp.float32),
               jax.ShapeDtypeStruct((A, 1), jnp.float32),
               jax.ShapeDtypeStruct((A, 1), jnp.float32)],
    compiler_params=pltpu.CompilerParams(vmem_limit_bytes=100 * 1024 * 1024),
)


EROWS = A // NS // 128
SCOLS = S // NS


def _norm_body(e0, e1, eh, soa, ngs_o, tgs_o, hyb_o,
               soa_v, e_v, out_v, tbl_v, part_v, comb_v, denom_v,
               tbl_sh, denom_sh):
    cid = lax.axis_index("c")
    sid = lax.axis_index("s")
    lanes = lax.iota(jnp.int32, L)
    rbase = sid * EROWS
    zeros16 = jnp.zeros((L,), jnp.float32)

    pltpu.sync_copy(soa.at[pl.ds(rbase, EROWS)], soa_v)

    def expert(e_hbm, out_hbm):
        pltpu.sync_copy(e_hbm.at[pl.ds(rbase, EROWS)], e_v)

        def ztbl(i, carry):
            tbl_v[pl.ds(i * L, L)] = zeros16
            return carry

        lax.fori_loop(0, S // L, ztbl, 0)
        for r in range(EROWS):
            for c in range(D // L):
                idx = soa_v[r, pl.ds(c * L, L)]
                val = e_v[r, pl.ds(c * L, L)]
                for lane in range(L):
                    plsc.addupdate_scatter(tbl_v, [idx], val,
                                           mask=lanes == lane)
        pltpu.sync_copy(tbl_v, tbl_sh.at[sid])
        plsc.subcore_barrier()

        colbase = sid * SCOLS
        pltpu.sync_copy(tbl_sh.at[:, pl.ds(colbase, SCOLS)], part_v)
        for g in range(SCOLS // L):
            acc = part_v[0, pl.ds(g * L, L)]
            for rr in range(1, NS):
                acc = acc + part_v[rr, pl.ds(g * L, L)]
            comb_v[pl.ds(g * L, L)] = acc
        pltpu.sync_copy(comb_v, denom_sh.at[pl.ds(colbase, SCOLS)])
        plsc.subcore_barrier()

        pltpu.sync_copy(denom_sh, denom_v)
        for r in range(EROWS):
            for c in range(D // L):
                idx = soa_v[r, pl.ds(c * L, L)]
                val = e_v[r, pl.ds(c * L, L)]
                d = plsc.load_gather(denom_v, [idx])
                out_v[r, pl.ds(c * L, L)] = val / d
        pltpu.sync_copy(out_v, out_hbm.at[pl.ds(rbase, EROWS)])
        plsc.subcore_barrier()

    @pl.when(cid == 0)
    def _():
        expert(e0, ngs_o)
        expert(eh, hyb_o)

    @pl.when(cid == 1)
    def _():
        expert(e1, tgs_o)


_norm = pl.kernel(
    _norm_body,
    out_type=[jax.ShapeDtypeStruct((A // 128, 128), jnp.float32),
              jax.ShapeDtypeStruct((A // 128, 128), jnp.float32),
              jax.ShapeDtypeStruct((A // 128, 128), jnp.float32)],
    mesh=plsc.VectorSubcoreMesh(core_axis_name="c", subcore_axis_name="s"),
    scratch_types=[
        pltpu.VMEM((EROWS, 128), jnp.int32),
        pltpu.VMEM((EROWS, 128), jnp.float32),
        pltpu.VMEM((EROWS, 128), jnp.float32),
        pltpu.VMEM((S,), jnp.float32),
        pltpu.VMEM((NS, SCOLS), jnp.float32),
        pltpu.VMEM((SCOLS,), jnp.float32),
        pltpu.VMEM((S,), jnp.float32),
        pltpu.VMEM_SHARED((NS, S), jnp.float32),
        pltpu.VMEM_SHARED((S,), jnp.float32),
    ],
    compiler_params=pltpu.CompilerParams(needs_layout_passes=False),
)


def kernel(x0, x1, allele_ids0, allele_ids1, site_ids0, site_ids1,
           site_of_allele, W0, b0, W1, b1, Wm, bm, We0, be0, We1, be1,
           We2, be2):
    b0r = b0.reshape(1, D)
    b1r = b1.reshape(1, D)
    sid0 = site_ids0.astype(jnp.int32).reshape(N // CH, CH)
    aid0 = allele_ids0.astype(jnp.int32).reshape(N // CH, CH)
    sid1 = site_ids1.astype(jnp.int32).reshape(N // CH, CH)
    aid1 = allele_ids1.astype(jnp.int32).reshape(N // CH, CH)

    r0a, r1a = _conv_a(x0, W0, b0r, x1, W1, b1r)
    r0b, r1b = _conv_b(x0, W0, b0r, x1, W1, b1r)
    pa = _segsum_a(r0a, r1a, sid0, aid0, sid1, aid1)
    pb = _segsum_b(r0b, r1b, sid0, aid0, sid1, aid1)

    meta, e0, e1, eh = _heads(*pa, *pb,
                              Wm[:D], Wm[D:], bm.reshape(1, 3),
                              We0, be0.reshape(1, 1),
                              We1, be1.reshape(1, 1),
                              We2[:D], We2[D:], be2.reshape(1, 1))

    soa = site_of_allele.astype(jnp.int32).reshape(A // 128, 128)
    ngs, tgs, hyb = _norm(e0.reshape(A // 128, 128),
                          e1.reshape(A // 128, 128),
                          eh.reshape(A // 128, 128), soa)
    return ngs.reshape(A), tgs.reshape(A), hyb.reshape(A), meta


# speedup vs baseline: 1.0252x; 1.0252x over previous
import jax
import jax.numpy as jnp
from jax import lax
from jax.experimental import pallas as pl
from jax.experimental.pallas import tpu as pltpu
from jax.experimental.pallas import tpu_sc as plsc

N = 131072
S = 4096
A = 8192
D = 128
NS = 16
L = 16

ROWS_PER_SUB = N // NS
CH = 64
NCHUNK = ROWS_PER_SUB // CH
IDROWS = ROWS_PER_SUB // 128

BN = 2048

_HI = lax.Precision.DEFAULT


def _mm(x, w):
    return lax.dot_general(x, w, (((1,), (0,)), ((), ())),
                           precision=_HI, preferred_element_type=jnp.float32)


def _conv_body(x0_ref, w0_ref, b0_ref, x1_ref, w1_ref, b1_ref, r0_ref, r1_ref):
    r0_ref[...] = jnp.maximum(_mm(x0_ref[...], w0_ref[...]) + b0_ref[...], 0.0)
    r1_ref[...] = jnp.maximum(_mm(x1_ref[...], w1_ref[...]) + b1_ref[...], 0.0)


def _make_conv(nrows, row0):
    o = row0 // BN
    return pl.pallas_call(
        _conv_body,
        grid=(nrows // BN,),
        in_specs=[
            pl.BlockSpec((BN, D), lambda i, o=o: (i + o, 0)),
            pl.BlockSpec((D, D), lambda i: (0, 0)),
            pl.BlockSpec((1, D), lambda i: (0, 0)),
            pl.BlockSpec((BN, D), lambda i, o=o: (i + o, 0)),
            pl.BlockSpec((D, D), lambda i: (0, 0)),
            pl.BlockSpec((1, D), lambda i: (0, 0)),
        ],
        out_specs=[pl.BlockSpec((BN, D), lambda i: (i, 0)),
                   pl.BlockSpec((BN, D), lambda i: (i, 0))],
        out_shape=[jax.ShapeDtypeStruct((nrows, D), jnp.float32),
                   jax.ShapeDtypeStruct((nrows, D), jnp.float32)],
    )


NA = N // 2
NB = N - NA

_conv_a = _make_conv(NA, 0)
_conv_b = _make_conv(NB, NA)


def _make_segsum(nrows, idrow0):
    rows_per_sub = nrows // NS
    nchunk = rows_per_sub // CH

    def body(*refs):
        (r0, r1, sid0, aid0, sid1, aid1,
         rs0_o, rs1_o, ra0_o, ra1_o,
         buf0, buf1, sidv, aidv, ibuf, ibuf2, midx, mbuf,
         site_acc, all_acc, idinfo,
         gsem0, gsem1, ssem0, ssem1) = refs
        cid = lax.axis_index("c")
        sid = lax.axis_index("s")
        ssl = pl.ds(sid * (S // NS), S // NS)
        asl = pl.ds(sid * (A // NS), A // NS)
        lanes = lax.iota(jnp.int32, L)
        zeros16 = jnp.zeros((L,), jnp.float32)

        def zrow(i, carry):
            for c in range(D // L):
                buf0[i, pl.ds(c * L, L)] = zeros16
            return carry

        lax.fori_loop(0, CH, zrow, 0)

        for t in range(S // NS // CH):
            pltpu.sync_copy(
                buf0, site_acc.at[pl.ds(sid * (S // NS) + t * CH, CH)])
        for t in range(A // NS // CH):
            pltpu.sync_copy(
                buf0, all_acc.at[pl.ds(sid * (A // NS) + t * CH, CH)])

        pltpu.sync_copy(buf0.at[pl.ds(0, 1)], site_acc.at[pl.ds(S + sid, 1)])
        pltpu.sync_copy(buf0.at[pl.ds(0, 1)],
                        site_acc.at[pl.ds(S + NS + sid, 1)])
        pltpu.sync_copy(buf0.at[pl.ds(0, 1)], all_acc.at[pl.ds(A + sid, 1)])
        pltpu.sync_copy(buf0.at[pl.ds(0, 1)],
                        all_acc.at[pl.ds(A + NS + sid, 1)])

        def run(r_hbm, s_hbm, a_hbm):
            base_row = sid * rows_per_sub
            idbase = idrow0 + sid * nchunk
            pltpu.sync_copy(s_hbm.at[pl.ds(idbase, nchunk)], sidv)
            pltpu.sync_copy(a_hbm.at[pl.ds(idbase, nchunk)], aidv)

            first_s = sidv[0, pl.ds(0, L)][0]
            last_s = sidv[nchunk - 1, pl.ds(CH - L, L)][L - 1]
            first_a = aidv[0, pl.ds(0, L)][0]
            last_a = aidv[nchunk - 1, pl.ds(CH - L, L)][L - 1]
            info = jnp.where(lanes == 0, first_s,
                             jnp.where(lanes == 1, last_s,
                                       jnp.where(lanes == 2, first_a,
                                                 jnp.where(lanes == 3,
                                                           last_a, 0))))
            ibuf[...] = info
            pltpu.sync_copy(ibuf, idinfo.at[sid])

            def redirect(i, carry):
                for c in range(CH // L):
                    slc = pl.ds(c * L, L)
                    v = sidv[i, slc]
                    sidv[i, slc] = jnp.where(
                        v == first_s, S + sid,
                        jnp.where(v == last_s, S + NS + sid, v))
                    w = aidv[i, slc]
                    aidv[i, slc] = jnp.where(
                        w == first_a, A + sid,
                        jnp.where(w == last_a, A + NS + sid, w))
                return carry

            lax.fori_loop(0, nchunk, redirect, 0)

            bufs = (buf0, buf1)
            gsems = (gsem0, gsem1)
            ssems = (ssem0, ssem1)

            pltpu.async_copy(r_hbm.at[pl.ds(base_row, CH)], buf0, gsem0)
            pltpu.async_copy(r_hbm.at[pl.ds(base_row + CH, CH)], buf1, gsem1)

            def loop(i, carry):
                for b in range(2):
                    j = 2 * i + b
                    buf, gsem, ssem = bufs[b], gsems[b], ssems[b]
                    pltpu.make_async_copy(
                        r_hbm.at[pl.ds(base_row, CH)], buf, gsem).wait()
                    sidx = sidv.at[j]
                    aidx = aidv.at[j]
                    sa = pltpu.async_copy(buf, site_acc.at[sidx], ssem,
                                          add=True)
                    sb = pltpu.async_copy(buf, all_acc.at[aidx], ssem,
                                          add=True)
                    sa.wait()
                    sb.wait()

                    @pl.when(j + 2 < nchunk)
                    def _():
                        pltpu.async_copy(
                            r_hbm.at[pl.ds(base_row + (j + 2) * CH, CH)],
                            buf, gsem)
                return carry

            lax.fori_loop(0, nchunk // 2, loop, 0)

        @pl.when(cid == 0)
        def _():
            run(r0, sid0, aid0)

        @pl.when(cid == 1)
        def _():
            run(r1, sid1, aid1)

        plsc.subcore_barrier()

        @pl.when(sid == 0)
        def _():
            pltpu.sync_copy(idinfo, ibuf2)
            for acc, base, c0 in ((site_acc, S, 0), (all_acc, A, 2)):
                for blk, c in ((0, c0), (NS, c0 + 1)):
                    tgt = plsc.load_gather(
                        ibuf2, [lanes, jnp.full((L,), c, jnp.int32)])
                    midx[...] = tgt
                    pltpu.sync_copy(acc.at[pl.ds(base + blk, NS)], mbuf)
                    pltpu.sync_copy(mbuf, acc.at[midx], add=True)

        plsc.subcore_barrier()

        def flush(acc, out, sl):
            pltpu.sync_copy(acc.at[sl], out.at[sl])

        @pl.when(cid == 0)
        def _():
            flush(site_acc, rs0_o, ssl)
            flush(all_acc, ra0_o, asl)

        @pl.when(cid == 1)
        def _():
            flush(site_acc, rs1_o, ssl)
            flush(all_acc, ra1_o, asl)

    return pl.kernel(
        body,
        out_type=[jax.ShapeDtypeStruct((S, D), jnp.float32),
                  jax.ShapeDtypeStruct((S, D), jnp.float32),
                  jax.ShapeDtypeStruct((A, D), jnp.float32),
                  jax.ShapeDtypeStruct((A, D), jnp.float32)],
        mesh=plsc.VectorSubcoreMesh(core_axis_name="c", subcore_axis_name="s"),
        scratch_types=[
            pltpu.VMEM((CH, D), jnp.float32),
            pltpu.VMEM((CH, D), jnp.float32),
            pltpu.VMEM((nchunk, CH), jnp.int32),
            pltpu.VMEM((nchunk, CH), jnp.int32),
            pltpu.VMEM((L,), jnp.int32),
            pltpu.VMEM((NS, L), jnp.int32),
            pltpu.VMEM((L,), jnp.int32),
            pltpu.VMEM((NS, D), jnp.float32),
            pltpu.VMEM_SHARED((S + 2 * NS, D), jnp.float32),
            pltpu.VMEM_SHARED((A + 2 * NS, D), jnp.float32),
            pltpu.VMEM_SHARED((NS, L), jnp.int32),
            pltpu.SemaphoreType.DMA,
            pltpu.SemaphoreType.DMA,
            pltpu.SemaphoreType.DMA,
            pltpu.SemaphoreType.DMA,
        ],
        compiler_params=pltpu.CompilerParams(needs_layout_passes=False,
                                             use_tc_tiling_on_sc=False),
    )


_segsum_a = _make_segsum(NA, 0)
_segsum_b = _make_segsum(NB, NA // CH)


def _heads_body(rs0a, rs1a, ra0a, ra1a, rs0b, rs1b, ra0b, ra1b,
                wm0, wm1, bm, we0, be0, we1, be1,
                w2a, w2b, be2, meta_o, e0_o, e1_o, eh_o):
    rs0 = rs0a[...] + rs0b[...]
    rs1 = rs1a[...] + rs1b[...]
    ra0 = ra0a[...] + ra0b[...]
    ra1 = ra1a[...] + ra1b[...]
    m = _mm(rs0, wm0[...]) + _mm(rs1, wm1[...]) + bm[...]
    m = m - jnp.max(m, axis=1, keepdims=True)
    p = jnp.exp(m)
    meta_o[...] = p / jnp.sum(p, axis=1, keepdims=True)

    s0 = _mm(ra0, we0[...]) + be0[...]
    e0_o[...] = jnp.exp(s0 - jnp.max(s0))
    s1 = _mm(ra1, we1[...]) + be1[...]
    e1_o[...] = jnp.exp(s1 - jnp.max(s1))
    sh = _mm(ra0, w2a[...]) + _mm(ra1, w2b[...]) + be2[...]
    eh_o[...] = jnp.exp(sh - jnp.max(sh))


_heads = pl.pallas_call(
    _heads_body,
    out_shape=[jax.ShapeDtypeStruct((S, 3), jnp.float32),
               jax.ShapeDtypeStruct((A, 1), jnp.float32),
               jax.ShapeDtypeStruct((A, 1), jnp.float32),
               jax.ShapeDtypeStruct((A, 1), jnp.float32)],
    compiler_params=pltpu.CompilerParams(vmem_limit_bytes=100 * 1024 * 1024),
)


EROWS = A // NS // 128
SCOLS = S // NS


def _norm_body(e0, e1, eh, soa, ngs_o, tgs_o, hyb_o,
               soa_v, e_v, out_v, tbl_v, part_v, comb_v, denom_v,
               tbl_sh, denom_sh):
    cid = lax.axis_index("c")
    sid = lax.axis_index("s")
    lanes = lax.iota(jnp.int32, L)
    rbase = sid * EROWS
    zeros16 = jnp.zeros((L,), jnp.float32)

    pltpu.sync_copy(soa.at[pl.ds(rbase, EROWS)], soa_v)

    def expert(e_hbm, out_hbm):
        pltpu.sync_copy(e_hbm.at[pl.ds(rbase, EROWS)], e_v)

        def ztbl(i, carry):
            tbl_v[pl.ds(i * L, L)] = zeros16
            return carry

        lax.fori_loop(0, S // L, ztbl, 0)
        for r in range(EROWS):
            for c in range(D // L):
                idx = soa_v[r, pl.ds(c * L, L)]
                val = e_v[r, pl.ds(c * L, L)]
                for lane in range(L):
                    plsc.addupdate_scatter(tbl_v, [idx], val,
                                           mask=lanes == lane)
        pltpu.sync_copy(tbl_v, tbl_sh.at[sid])
        plsc.subcore_barrier()

        colbase = sid * SCOLS
        pltpu.sync_copy(tbl_sh.at[:, pl.ds(colbase, SCOLS)], part_v)
        for g in range(SCOLS // L):
            acc = part_v[0, pl.ds(g * L, L)]
            for rr in range(1, NS):
                acc = acc + part_v[rr, pl.ds(g * L, L)]
            comb_v[pl.ds(g * L, L)] = acc
        pltpu.sync_copy(comb_v, denom_sh.at[pl.ds(colbase, SCOLS)])
        plsc.subcore_barrier()

        pltpu.sync_copy(denom_sh, denom_v)
        for r in range(EROWS):
            for c in range(D // L):
                idx = soa_v[r, pl.ds(c * L, L)]
                val = e_v[r, pl.ds(c * L, L)]
                d = plsc.load_gather(denom_v, [idx])
                out_v[r, pl.ds(c * L, L)] = val / d
        pltpu.sync_copy(out_v, out_hbm.at[pl.ds(rbase, EROWS)])
        plsc.subcore_barrier()

    @pl.when(cid == 0)
    def _():
        expert(e0, ngs_o)
        expert(eh, hyb_o)

    @pl.when(cid == 1)
    def _():
        expert(e1, tgs_o)


_norm = pl.kernel(
    _norm_body,
    out_type=[jax.ShapeDtypeStruct((A // 128, 128), jnp.float32),
              jax.ShapeDtypeStruct((A // 128, 128), jnp.float32),
              jax.ShapeDtypeStruct((A // 128, 128), jnp.float32)],
    mesh=plsc.VectorSubcoreMesh(core_axis_name="c", subcore_axis_name="s"),
    scratch_types=[
        pltpu.VMEM((EROWS, 128), jnp.int32),
        pltpu.VMEM((EROWS, 128), jnp.float32),
        pltpu.VMEM((EROWS, 128), jnp.float32),
        pltpu.VMEM((S,), jnp.float32),
        pltpu.VMEM((NS, SCOLS), jnp.float32),
        pltpu.VMEM((SCOLS,), jnp.float32),
        pltpu.VMEM((S,), jnp.float32),
        pltpu.VMEM_SHARED((NS, S), jnp.float32),
        pltpu.VMEM_SHARED((S,), jnp.float32),
    ],
    compiler_params=pltpu.CompilerParams(needs_layout_passes=False),
)


def kernel(x0, x1, allele_ids0, allele_ids1, site_ids0, site_ids1,
           site_of_allele, W0, b0, W1, b1, Wm, bm, We0, be0, We1, be1,
           We2, be2):
    b0r = b0.reshape(1, D)
    b1r = b1.reshape(1, D)
    sid0 = site_ids0.astype(jnp.int32).reshape(N // CH, CH)
    aid0 = allele_ids0.astype(jnp.int32).reshape(N // CH, CH)
    sid1 = site_ids1.astype(jnp.int32).reshape(N // CH, CH)
    aid1 = allele_ids1.astype(jnp.int32).reshape(N // CH, CH)

    r0a, r1a = _conv_a(x0, W0, b0r, x1, W1, b1r)
    r0b, r1b = _conv_b(x0, W0, b0r, x1, W1, b1r)
    pa = _segsum_a(r0a, r1a, sid0, aid0, sid1, aid1)
    pb = _segsum_b(r0b, r1b, sid0, aid0, sid1, aid1)

    meta, e0, e1, eh = _heads(*pa, *pb,
                              Wm[:D], Wm[D:], bm.reshape(1, 3),
                              We0, be0.reshape(1, 1),
                              We1, be1.reshape(1, 1),
                              We2[:D], We2[D:], be2.reshape(1, 1))

    soa = site_of_allele.astype(jnp.int32).reshape(A // 128, 128)
    ngs, tgs, hyb = _norm(e0.reshape(A // 128, 128),
                          e1.reshape(A // 128, 128),
                          eh.reshape(A // 128, 128), soa)
    return ngs.reshape(A), tgs.reshape(A), hyb.reshape(A), meta
